# baseline (speedup 1.0000x reference)
import functools

import jax
import jax.numpy as jnp
from jax import lax
from jax.experimental import pallas as pl
from jax.experimental.pallas import tpu as pltpu
from jax.experimental.pallas import tpu_sc as plsc

VOCAB = 1000000
EMBED_DIM = 32
BATCH = 4096
HIST = 200

_NC = 2
_NS = 16
_NW = _NC * _NS
_BBLK = BATCH // _NW
_VROWS = VOCAB * EMBED_DIM // 128
_HCHUNK = 40


def _make_kernel():
    mesh = plsc.VectorSubcoreMesh(core_axis_name="c", subcore_axis_name="s")

    @functools.partial(
        pl.kernel,
        out_type=jax.ShapeDtypeStruct((HIST, EMBED_DIM, BATCH), jnp.float32),
        mesh=mesh,
        scratch_types=[
            pltpu.VMEM((_HCHUNK, _BBLK), jnp.int32),
            pltpu.VMEM((_HCHUNK, _BBLK), jnp.int32),
            pltpu.VMEM((4, _BBLK, 128), jnp.float32),
            pltpu.VMEM((2, EMBED_DIM, _BBLK), jnp.float32),
            pltpu.SemaphoreType.DMA,
            pltpu.SemaphoreType.DMA,
            pltpu.SemaphoreType.DMA,
            pltpu.SemaphoreType.DMA,
            pltpu.SemaphoreType.DMA,
            pltpu.SemaphoreType.DMA,
        ],
        compiler_params=pltpu.CompilerParams(use_tc_tiling_on_sc=True,
                                             needs_layout_passes=False),
    )
    def emb_kernel(table4_hbm, idxT_hbm, out_hbm, idx_v, q_v, g_v, o_v,
                   sem_g0, sem_g1, sem_g2, sem_g3, sem_w0, sem_w1):
        wid = lax.axis_index("s") * _NC + lax.axis_index("c")
        b0 = wid * _BBLK
        sem_g = (sem_g0, sem_g1, sem_g2, sem_g3)
        sem_w = (sem_w0, sem_w1)

        lane = lax.iota(jnp.int32, 16)

        def start_gather(h, buf):
            pltpu.async_copy(table4_hbm.at[q_v.at[h]], g_v.at[buf],
                             sem_g[buf])

        def wait_gather(buf):
            pltpu.make_async_copy(table4_hbm.at[q_v.at[0]], g_v.at[buf],
                                  sem_g[buf]).wait()

        def start_write(hg, buf):
            pltpu.async_copy(o_v.at[buf], out_hbm.at[hg, :, pl.ds(b0, _BBLK)],
                             sem_w[buf])

        def wait_write(buf):
            pltpu.make_async_copy(o_v.at[buf],
                                  out_hbm.at[0, :, pl.ds(b0, _BBLK)],
                                  sem_w[buf]).wait()

        def extract(h, buf, obuf):
            @pl.loop(0, _BBLK // 16)
            def _(bg):
                j = bg * 16
                r16 = idx_v[h, pl.ds(j, 16)]
                col0 = (r16 & 3) * EMBED_DIM
                row = lane + j
                for d0 in range(0, EMBED_DIM, 8):
                    vecs = [
                        plsc.load_gather(g_v.at[buf], [row, col0 + (d0 + i)])
                        for i in range(8)
                    ]
                    for i in range(8):
                        o_v[obuf, d0 + i, pl.ds(j, 16)] = vecs[i]

        for c in range(HIST // _HCHUNK):
            hc0 = c * _HCHUNK
            pltpu.sync_copy(idxT_hbm.at[pl.ds(hc0, _HCHUNK),
                                        pl.ds(b0, _BBLK)], idx_v)

            @pl.loop(0, _HCHUNK * _BBLK // 16)
            def _(i):
                h = i // (_BBLK // 16)
                j = (i % (_BBLK // 16)) * 16
                q_v[h, pl.ds(j, 16)] = lax.shift_right_logical(
                    idx_v[h, pl.ds(j, 16)], 2)

            start_gather(0, 0)
            start_gather(1, 1)
            start_gather(2, 2)

            @pl.loop(0, _HCHUNK // 4)
            def _(i):
                h = i * 4
                for k in range(4):
                    hk = h + k

                    @pl.when(hk + 3 < _HCHUNK)
                    def _():
                        start_gather(hk + 3, (k + 3) % 4)

                    wait_gather(k)

                    @pl.when((c > 0) | (hk >= 2))
                    def _():
                        wait_write(k % 2)

                    extract(hk, k, k % 2)
                    start_write(hc0 + hk, k % 2)

        wait_write(0)
        wait_write(1)

    return emb_kernel


_emb_kernel = _make_kernel()


@jax.jit
def kernel(inputs, embedding_matrix):
    table4 = embedding_matrix.reshape(_VROWS, 128)
    idx_t = inputs.T.astype(jnp.int32)
    out_t = _emb_kernel(table4, idx_t)
    return jnp.transpose(out_t, (2, 0, 1))


# speedup vs baseline: 1.0080x; 1.0080x over previous
import functools

import jax
import jax.numpy as jnp
from jax import lax
from jax.experimental import pallas as pl
from jax.experimental.pallas import tpu as pltpu
from jax.experimental.pallas import tpu_sc as plsc

VOCAB = 1000000
EMBED_DIM = 32
BATCH = 4096
HIST = 200

_NC = 2
_NS = 16
_NW = _NC * _NS
_BBLK = BATCH // _NW
_VROWS = VOCAB * EMBED_DIM // 128
_HCHUNK = 40


def _make_kernel():
    mesh = plsc.VectorSubcoreMesh(core_axis_name="c", subcore_axis_name="s")

    @functools.partial(
        pl.kernel,
        out_type=jax.ShapeDtypeStruct((HIST, EMBED_DIM, BATCH), jnp.float32),
        mesh=mesh,
        scratch_types=[
            pltpu.VMEM((_HCHUNK, _BBLK), jnp.int32),
            pltpu.VMEM((_HCHUNK, _BBLK), jnp.int32),
            pltpu.VMEM((4, _BBLK, 128), jnp.float32),
            pltpu.VMEM((2, EMBED_DIM, _BBLK), jnp.float32),
            pltpu.SemaphoreType.DMA,
            pltpu.SemaphoreType.DMA,
            pltpu.SemaphoreType.DMA,
            pltpu.SemaphoreType.DMA,
            pltpu.SemaphoreType.DMA,
            pltpu.SemaphoreType.DMA,
        ],
        compiler_params=pltpu.CompilerParams(use_tc_tiling_on_sc=True,
                                             needs_layout_passes=False),
    )
    def emb_kernel(table4_hbm, idxT_hbm, out_hbm, idx_v, q_v, g_v, o_v,
                   sem_g0, sem_g1, sem_g2, sem_g3, sem_w0, sem_w1):
        wid = lax.axis_index("s") * _NC + lax.axis_index("c")
        b0 = wid * _BBLK
        sem_g = (sem_g0, sem_g1, sem_g2, sem_g3)
        sem_w = (sem_w0, sem_w1)

        lane = lax.iota(jnp.int32, 16)

        def start_gather(h, buf):
            pltpu.async_copy(table4_hbm.at[q_v.at[h]], g_v.at[buf],
                             sem_g[buf])

        def wait_gather(buf):
            pltpu.make_async_copy(table4_hbm.at[q_v.at[0]], g_v.at[buf],
                                  sem_g[buf]).wait()

        def start_write(hg, buf):
            pltpu.async_copy(o_v.at[buf], out_hbm.at[hg, :, pl.ds(b0, _BBLK)],
                             sem_w[buf])

        def wait_write(buf):
            pltpu.make_async_copy(o_v.at[buf],
                                  out_hbm.at[0, :, pl.ds(b0, _BBLK)],
                                  sem_w[buf]).wait()

        def extract(h, buf, obuf):
            @pl.loop(0, _BBLK // 16)
            def _(bg):
                j = bg * 16
                r16 = idx_v[h, pl.ds(j, 16)]
                col0 = (r16 & 3) * EMBED_DIM
                row = lane + j
                for d0 in range(0, EMBED_DIM, 16):
                    vecs = [
                        plsc.load_gather(g_v.at[buf], [row, col0 + (d0 + i)])
                        for i in range(16)
                    ]
                    for i in range(16):
                        o_v[obuf, d0 + i, pl.ds(j, 16)] = vecs[i]

        for c in range(HIST // _HCHUNK):
            hc0 = c * _HCHUNK
            pltpu.sync_copy(idxT_hbm.at[pl.ds(hc0, _HCHUNK),
                                        pl.ds(b0, _BBLK)], idx_v)

            @pl.loop(0, _HCHUNK * _BBLK // 16)
            def _(i):
                h = i // (_BBLK // 16)
                j = (i % (_BBLK // 16)) * 16
                q_v[h, pl.ds(j, 16)] = lax.shift_right_logical(
                    idx_v[h, pl.ds(j, 16)], 2)

            start_gather(0, 0)
            start_gather(1, 1)
            start_gather(2, 2)

            @pl.loop(0, _HCHUNK // 4)
            def _(i):
                h = i * 4
                for k in range(4):
                    hk = h + k

                    @pl.when(hk + 3 < _HCHUNK)
                    def _():
                        start_gather(hk + 3, (k + 3) % 4)

                    wait_gather(k)

                    @pl.when((c > 0) | (hk >= 2))
                    def _():
                        wait_write(k % 2)

                    extract(hk, k, k % 2)
                    start_write(hc0 + hk, k % 2)

        wait_write(0)
        wait_write(1)

    return emb_kernel


_emb_kernel = _make_kernel()


@jax.jit
def kernel(inputs, embedding_matrix):
    table4 = embedding_matrix.reshape(_VROWS, 128)
    idx_t = inputs.T.astype(jnp.int32)
    out_t = _emb_kernel(table4, idx_t)
    return jnp.transpose(out_t, (2, 0, 1))


# speedup vs baseline: 1.0088x; 1.0008x over previous
import functools

import jax
import jax.numpy as jnp
from jax import lax
from jax.experimental import pallas as pl
from jax.experimental.pallas import tpu as pltpu
from jax.experimental.pallas import tpu_sc as plsc

VOCAB = 1000000
EMBED_DIM = 32
BATCH = 4096
HIST = 200

_NC = 2
_NS = 16
_NW = _NC * _NS
_BBLK = BATCH // _NW
_VROWS = VOCAB * EMBED_DIM // 128
_HCHUNK = 40


def _make_kernel():
    mesh = plsc.VectorSubcoreMesh(core_axis_name="c", subcore_axis_name="s")

    @functools.partial(
        pl.kernel,
        out_type=jax.ShapeDtypeStruct((HIST, EMBED_DIM, BATCH), jnp.float32),
        mesh=mesh,
        scratch_types=[
            pltpu.VMEM((_HCHUNK, _BBLK), jnp.int32),
            pltpu.VMEM((_HCHUNK, _BBLK), jnp.int32),
            pltpu.VMEM((4, _BBLK, 128), jnp.float32),
            pltpu.VMEM((2, EMBED_DIM, _BBLK), jnp.float32),
            pltpu.SemaphoreType.DMA,
            pltpu.SemaphoreType.DMA,
            pltpu.SemaphoreType.DMA,
            pltpu.SemaphoreType.DMA,
            pltpu.SemaphoreType.DMA,
            pltpu.SemaphoreType.DMA,
        ],
        compiler_params=pltpu.CompilerParams(use_tc_tiling_on_sc=True,
                                             needs_layout_passes=False,
                                             disable_bounds_checks=True),
    )
    def emb_kernel(table4_hbm, idxT_hbm, out_hbm, idx_v, q_v, g_v, o_v,
                   sem_g0, sem_g1, sem_g2, sem_g3, sem_w0, sem_w1):
        wid = lax.axis_index("s") * _NC + lax.axis_index("c")
        b0 = wid * _BBLK
        sem_g = (sem_g0, sem_g1, sem_g2, sem_g3)
        sem_w = (sem_w0, sem_w1)

        lane = lax.iota(jnp.int32, 16)

        def start_gather(h, buf):
            pltpu.async_copy(table4_hbm.at[q_v.at[h]], g_v.at[buf],
                             sem_g[buf])

        def wait_gather(buf):
            pltpu.make_async_copy(table4_hbm.at[q_v.at[0]], g_v.at[buf],
                                  sem_g[buf]).wait()

        def start_write(hg, buf):
            pltpu.async_copy(o_v.at[buf], out_hbm.at[hg, :, pl.ds(b0, _BBLK)],
                             sem_w[buf])

        def wait_write(buf):
            pltpu.make_async_copy(o_v.at[buf],
                                  out_hbm.at[0, :, pl.ds(b0, _BBLK)],
                                  sem_w[buf]).wait()

        def extract(h, buf, obuf):
            @pl.loop(0, _BBLK // 16)
            def _(bg):
                j = bg * 16
                r16 = idx_v[h, pl.ds(j, 16)]
                col0 = (r16 & 3) * EMBED_DIM
                row = lane + j
                for d0 in range(0, EMBED_DIM, 16):
                    vecs = [
                        plsc.load_gather(g_v.at[buf], [row, col0 + (d0 + i)])
                        for i in range(16)
                    ]
                    for i in range(16):
                        o_v[obuf, d0 + i, pl.ds(j, 16)] = vecs[i]

        for c in range(HIST // _HCHUNK):
            hc0 = c * _HCHUNK
            pltpu.sync_copy(idxT_hbm.at[pl.ds(hc0, _HCHUNK),
                                        pl.ds(b0, _BBLK)], idx_v)

            @pl.loop(0, _HCHUNK * _BBLK // 16)
            def _(i):
                h = i // (_BBLK // 16)
                j = (i % (_BBLK // 16)) * 16
                q_v[h, pl.ds(j, 16)] = lax.shift_right_logical(
                    idx_v[h, pl.ds(j, 16)], 2)

            start_gather(0, 0)
            start_gather(1, 1)
            start_gather(2, 2)

            @pl.loop(0, _HCHUNK // 4)
            def _(i):
                h = i * 4
                for k in range(4):
                    hk = h + k

                    @pl.when(hk + 3 < _HCHUNK)
                    def _():
                        start_gather(hk + 3, (k + 3) % 4)

                    wait_gather(k)

                    @pl.when((c > 0) | (hk >= 2))
                    def _():
                        wait_write(k % 2)

                    extract(hk, k, k % 2)
                    start_write(hc0 + hk, k % 2)

        wait_write(0)
        wait_write(1)

    return emb_kernel


_emb_kernel = _make_kernel()


@jax.jit
def kernel(inputs, embedding_matrix):
    table4 = embedding_matrix.reshape(_VROWS, 128)
    idx_t = inputs.T.astype(jnp.int32)
    out_t = _emb_kernel(table4, idx_t)
    return jnp.transpose(out_t, (2, 0, 1))


# speedup vs baseline: 1.3033x; 1.2919x over previous
import functools

import jax
import jax.numpy as jnp
from jax import lax
from jax.experimental import pallas as pl
from jax.experimental.pallas import tpu as pltpu
from jax.experimental.pallas import tpu_sc as plsc

VOCAB = 1000000
EMBED_DIM = 32
BATCH = 4096
HIST = 200

_NC = 2
_NS = 16
_NW = _NC * _NS
_BBLK = BATCH // _NW
_VROWS = VOCAB * EMBED_DIM // 128
_HCHUNK = 40


def _make_kernel():
    mesh = plsc.VectorSubcoreMesh(core_axis_name="c", subcore_axis_name="s")

    @functools.partial(
        pl.kernel,
        out_type=jax.ShapeDtypeStruct((HIST, EMBED_DIM, BATCH), jnp.float32),
        mesh=mesh,
        scratch_types=[
            pltpu.VMEM((_HCHUNK, _BBLK), jnp.int32),
            pltpu.VMEM((_HCHUNK, _BBLK), jnp.int32),
            pltpu.VMEM((4, _BBLK, 128), jnp.float32),
            pltpu.VMEM((2, EMBED_DIM, _BBLK), jnp.float32),
            pltpu.SemaphoreType.DMA,
            pltpu.SemaphoreType.DMA,
            pltpu.SemaphoreType.DMA,
            pltpu.SemaphoreType.DMA,
            pltpu.SemaphoreType.DMA,
            pltpu.SemaphoreType.DMA,
        ],
        compiler_params=pltpu.CompilerParams(use_tc_tiling_on_sc=True,
                                             needs_layout_passes=False,
                                             disable_bounds_checks=True),
    )
    def emb_kernel(table4_hbm, idxT_hbm, out_hbm, idx_v, q_v, g_v, o_v,
                   sem_g0, sem_g1, sem_g2, sem_g3, sem_w0, sem_w1):
        wid = lax.axis_index("s") * _NC + lax.axis_index("c")
        b0 = wid * _BBLK
        sem_g = (sem_g0, sem_g1, sem_g2, sem_g3)
        sem_w = (sem_w0, sem_w1)

        lane = lax.iota(jnp.int32, 16)

        def start_gather(h, buf):
            pltpu.async_copy(table4_hbm.at[q_v.at[h]], g_v.at[buf],
                             sem_g[buf])

        def wait_gather(buf):
            pltpu.make_async_copy(table4_hbm.at[q_v.at[0]], g_v.at[buf],
                                  sem_g[buf]).wait()

        def start_write(hg, buf):
            pltpu.async_copy(o_v.at[buf], out_hbm.at[hg, :, pl.ds(b0, _BBLK)],
                             sem_w[buf])

        def wait_write(buf):
            pltpu.make_async_copy(o_v.at[buf],
                                  out_hbm.at[0, :, pl.ds(b0, _BBLK)],
                                  sem_w[buf]).wait()

        def extract(h, buf, obuf):
            @functools.partial(plsc.parallel_loop, 0, _BBLK // 16)
            def _(bg):
                j = bg * 16
                r16 = idx_v[h, pl.ds(j, 16)]
                col0 = (r16 & 3) * EMBED_DIM
                row = lane + j
                for d0 in range(0, EMBED_DIM, 16):
                    vecs = [
                        plsc.load_gather(g_v.at[buf], [row, col0 + (d0 + i)])
                        for i in range(16)
                    ]
                    for i in range(16):
                        o_v[obuf, d0 + i, pl.ds(j, 16)] = vecs[i]

        for c in range(HIST // _HCHUNK):
            hc0 = c * _HCHUNK
            pltpu.sync_copy(idxT_hbm.at[pl.ds(hc0, _HCHUNK),
                                        pl.ds(b0, _BBLK)], idx_v)

            @pl.loop(0, _HCHUNK * _BBLK // 16)
            def _(i):
                h = i // (_BBLK // 16)
                j = (i % (_BBLK // 16)) * 16
                q_v[h, pl.ds(j, 16)] = lax.shift_right_logical(
                    idx_v[h, pl.ds(j, 16)], 2)

            start_gather(0, 0)
            start_gather(1, 1)
            start_gather(2, 2)

            @pl.loop(0, _HCHUNK // 4)
            def _(i):
                h = i * 4
                for k in range(4):
                    hk = h + k

                    @pl.when(hk + 3 < _HCHUNK)
                    def _():
                        start_gather(hk + 3, (k + 3) % 4)

                    wait_gather(k)

                    @pl.when((c > 0) | (hk >= 2))
                    def _():
                        wait_write(k % 2)

                    extract(hk, k, k % 2)
                    start_write(hc0 + hk, k % 2)

        wait_write(0)
        wait_write(1)

    return emb_kernel


_emb_kernel = _make_kernel()


@jax.jit
def kernel(inputs, embedding_matrix):
    table4 = embedding_matrix.reshape(_VROWS, 128)
    idx_t = inputs.T.astype(jnp.int32)
    out_t = _emb_kernel(table4, idx_t)
    return jnp.transpose(out_t, (2, 0, 1))
